# baseline (speedup 1.0000x reference)
import functools

import jax
import jax.numpy as jnp
from jax.experimental import pallas as pl
from jax.experimental.pallas import tpu as pltpu


def _body(a_ref, x_ref, xn_ref, w1_ref, w2_ref, adj_ref, diff_ref,
          b1_ref, b2_ref,
          h1_ref, h2_ref, h3_ref, h4_ref, c1_ref, c2_ref,
          ya_ref, yb_ref, *, n):
    i = pl.program_id(0)
    steps = pl.num_programs(0)
    alpha = a_ref[0]
    h = h1_ref.shape[1]

    @pl.when(i == 0)
    def _():
        xb = x_ref[...]
        xnb = xn_ref[...]
        w1 = w1_ref[...]
        w2 = w2_ref[...]
        ya_ref[...] = jnp.concatenate(
            [jnp.dot(xb, w1, preferred_element_type=jnp.float32),
             jnp.dot(xnb, w1, preferred_element_type=jnp.float32)], axis=1)
        yb_ref[...] = jnp.concatenate(
            [jnp.dot(xb, w2, preferred_element_type=jnp.float32),
             jnp.dot(xnb, w2, preferred_element_type=jnp.float32)], axis=1)

    pa = jnp.dot(adj_ref[...], ya_ref[...], preferred_element_type=jnp.float32)
    pd = jnp.dot(diff_ref[...], yb_ref[...], preferred_element_type=jnp.float32)

    z1 = pa[:, :h] + b1_ref[...]
    z3 = pa[:, h:] + b1_ref[...]
    z2 = pd[:, :h] + b2_ref[...]
    z4 = pd[:, h:] + b2_ref[...]

    h1 = jnp.where(z1 >= 0, z1, alpha * z1)
    h2 = jnp.where(z2 >= 0, z2, alpha * z2)
    h3 = jnp.where(z3 >= 0, z3, alpha * z3)
    h4 = jnp.where(z4 >= 0, z4, alpha * z4)

    h1_ref[...] = h1
    h2_ref[...] = h2
    h3_ref[...] = h3
    h4_ref[...] = h4

    s1 = jnp.sum(h1, axis=0, keepdims=True)
    s2 = jnp.sum(h2, axis=0, keepdims=True)

    @pl.when(i == 0)
    def _():
        c1_ref[...] = s1
        c2_ref[...] = s2

    @pl.when(i > 0)
    def _():
        c1_ref[...] += s1
        c2_ref[...] += s2

    @pl.when(i == steps - 1)
    def _():
        c1_ref[...] = jax.nn.sigmoid(c1_ref[...] * (1.0 / n))
        c2_ref[...] = jax.nn.sigmoid(c2_ref[...] * (1.0 / n))


def kernel(x, x_neg, adj, diff, W1, W2, b1, b2, a, is_sparse):
    n, f = x.shape
    h = W1.shape[1]

    bn = 80
    while n % bn != 0:
        bn //= 2
    grid = (n // bn,)
    a2 = jnp.reshape(a, (1,)).astype(jnp.float32)
    b1r = jnp.reshape(b1, (1, h))
    b2r = jnp.reshape(b2, (1, h))

    h1, h2, h3, h4, c1, c2 = pl.pallas_call(
        functools.partial(_body, n=float(n)),
        grid=grid,
        in_specs=[
            pl.BlockSpec(memory_space=pltpu.SMEM),
            pl.BlockSpec((n, f), lambda i: (0, 0)),
            pl.BlockSpec((n, f), lambda i: (0, 0)),
            pl.BlockSpec((f, h), lambda i: (0, 0)),
            pl.BlockSpec((f, h), lambda i: (0, 0)),
            pl.BlockSpec((bn, n), lambda i: (i, 0)),
            pl.BlockSpec((bn, n), lambda i: (i, 0)),
            pl.BlockSpec((1, h), lambda i: (0, 0)),
            pl.BlockSpec((1, h), lambda i: (0, 0)),
        ],
        out_specs=[
            pl.BlockSpec((bn, h), lambda i: (i, 0)),
            pl.BlockSpec((bn, h), lambda i: (i, 0)),
            pl.BlockSpec((bn, h), lambda i: (i, 0)),
            pl.BlockSpec((bn, h), lambda i: (i, 0)),
            pl.BlockSpec((1, h), lambda i: (0, 0)),
            pl.BlockSpec((1, h), lambda i: (0, 0)),
        ],
        out_shape=[
            jax.ShapeDtypeStruct((n, h), jnp.float32),
            jax.ShapeDtypeStruct((n, h), jnp.float32),
            jax.ShapeDtypeStruct((n, h), jnp.float32),
            jax.ShapeDtypeStruct((n, h), jnp.float32),
            jax.ShapeDtypeStruct((1, h), jnp.float32),
            jax.ShapeDtypeStruct((1, h), jnp.float32),
        ],
        scratch_shapes=[
            pltpu.VMEM((n, 2 * h), jnp.float32),
            pltpu.VMEM((n, 2 * h), jnp.float32),
        ],
    )(a2, x, x_neg, W1, W2, adj, diff, b1r, b2r)

    return (c1[0], c2[0], h1, h2, h3, h4)


# speedup vs baseline: 1.1777x; 1.1777x over previous
import functools

import jax
import jax.numpy as jnp
from jax.experimental import pallas as pl
from jax.experimental.pallas import tpu as pltpu


def _feat_body(x_ref, xn_ref, w1_ref, w2_ref, ya_ref, yb_ref):
    xb = x_ref[...]
    xnb = xn_ref[...]
    w1 = w1_ref[...]
    w2 = w2_ref[...]
    ya_ref[...] = jnp.concatenate(
        [jnp.dot(xb, w1, preferred_element_type=jnp.float32),
         jnp.dot(xnb, w1, preferred_element_type=jnp.float32)], axis=1)
    yb_ref[...] = jnp.concatenate(
        [jnp.dot(xb, w2, preferred_element_type=jnp.float32),
         jnp.dot(xnb, w2, preferred_element_type=jnp.float32)], axis=1)


def _prop_body(a_ref, m_ref, y_ref, b_ref, hp_ref, hn_ref, c_ref, *, n):
    i = pl.program_id(0)
    steps = pl.num_programs(0)
    alpha = a_ref[0]
    h = hp_ref.shape[1]

    p = jnp.dot(m_ref[...], y_ref[...], preferred_element_type=jnp.float32)
    zp = p[:, :h] + b_ref[...]
    zn = p[:, h:] + b_ref[...]
    hp = jnp.where(zp >= 0, zp, alpha * zp)
    hn = jnp.where(zn >= 0, zn, alpha * zn)
    hp_ref[...] = hp
    hn_ref[...] = hn

    s = jnp.sum(hp, axis=0, keepdims=True)

    @pl.when(i == 0)
    def _():
        c_ref[...] = s

    @pl.when(i > 0)
    def _():
        c_ref[...] += s

    @pl.when(i == steps - 1)
    def _():
        c_ref[...] = jax.nn.sigmoid(c_ref[...] * (1.0 / n))


def _propagate(a2, mat, y, br, n, h, bn):
    grid = (n // bn,)
    return pl.pallas_call(
        functools.partial(_prop_body, n=float(n)),
        grid=grid,
        in_specs=[
            pl.BlockSpec(memory_space=pltpu.SMEM),
            pl.BlockSpec((bn, n), lambda i: (i, 0)),
            pl.BlockSpec((n, 2 * h), lambda i: (0, 0)),
            pl.BlockSpec((1, h), lambda i: (0, 0)),
        ],
        out_specs=[
            pl.BlockSpec((bn, h), lambda i: (i, 0)),
            pl.BlockSpec((bn, h), lambda i: (i, 0)),
            pl.BlockSpec((1, h), lambda i: (0, 0)),
        ],
        out_shape=[
            jax.ShapeDtypeStruct((n, h), jnp.float32),
            jax.ShapeDtypeStruct((n, h), jnp.float32),
            jax.ShapeDtypeStruct((1, h), jnp.float32),
        ],
    )(a2, mat, y, br)


def kernel(x, x_neg, adj, diff, W1, W2, b1, b2, a, is_sparse):
    n, f = x.shape
    h = W1.shape[1]

    fb = n // 10 if n % 10 == 0 else n
    ya, yb = pl.pallas_call(
        _feat_body,
        grid=(n // fb,),
        in_specs=[
            pl.BlockSpec((fb, f), lambda i: (i, 0)),
            pl.BlockSpec((fb, f), lambda i: (i, 0)),
            pl.BlockSpec((f, h), lambda i: (0, 0)),
            pl.BlockSpec((f, h), lambda i: (0, 0)),
        ],
        out_specs=[
            pl.BlockSpec((fb, 2 * h), lambda i: (i, 0)),
            pl.BlockSpec((fb, 2 * h), lambda i: (i, 0)),
        ],
        out_shape=[
            jax.ShapeDtypeStruct((n, 2 * h), jnp.float32),
            jax.ShapeDtypeStruct((n, 2 * h), jnp.float32),
        ],
    )(x, x_neg, W1, W2)

    bn = 400
    while n % bn != 0 or bn % 8 != 0:
        bn //= 2
    a2 = jnp.reshape(a, (1,)).astype(jnp.float32)
    b1r = jnp.reshape(b1, (1, h))
    b2r = jnp.reshape(b2, (1, h))

    h1, h3, c1 = _propagate(a2, adj, ya, b1r, n, h, bn)
    h2, h4, c2 = _propagate(a2, diff, yb, b2r, n, h, bn)

    return (c1[0], c2[0], h1, h2, h3, h4)


# speedup vs baseline: 1.2261x; 1.0411x over previous
import functools

import jax
import jax.numpy as jnp
from jax.experimental import pallas as pl
from jax.experimental.pallas import tpu as pltpu


def _prop_body(a_ref, x_ref, xn_ref, w_ref, m_ref, b_ref,
               hp_ref, hn_ref, c_ref, y_ref, *, n):
    i = pl.program_id(0)
    steps = pl.num_programs(0)
    alpha = a_ref[0]
    h = hp_ref.shape[1]

    @pl.when(i == 0)
    def _():
        w = w_ref[...]
        y_ref[:, :h] = jnp.dot(x_ref[...], w,
                               preferred_element_type=jnp.float32)
        y_ref[:, h:] = jnp.dot(xn_ref[...], w,
                               preferred_element_type=jnp.float32)

    p = jnp.dot(m_ref[...], y_ref[...], preferred_element_type=jnp.float32)
    zp = p[:, :h] + b_ref[...]
    zn = p[:, h:] + b_ref[...]
    hp = jnp.where(zp >= 0, zp, alpha * zp)
    hn = jnp.where(zn >= 0, zn, alpha * zn)
    hp_ref[...] = hp
    hn_ref[...] = hn

    s = jnp.sum(hp, axis=0, keepdims=True)

    @pl.when(i == 0)
    def _():
        c_ref[...] = s

    @pl.when(i > 0)
    def _():
        c_ref[...] += s

    @pl.when(i == steps - 1)
    def _():
        c_ref[...] = jax.nn.sigmoid(c_ref[...] * (1.0 / n))


def _propagate(a2, x, xn, w, mat, br, n, f, h, bn):
    return pl.pallas_call(
        functools.partial(_prop_body, n=float(n)),
        grid=(n // bn,),
        in_specs=[
            pl.BlockSpec(memory_space=pltpu.SMEM),
            pl.BlockSpec((n, f), lambda i: (0, 0)),
            pl.BlockSpec((n, f), lambda i: (0, 0)),
            pl.BlockSpec((f, h), lambda i: (0, 0)),
            pl.BlockSpec((bn, n), lambda i: (i, 0)),
            pl.BlockSpec((1, h), lambda i: (0, 0)),
        ],
        out_specs=[
            pl.BlockSpec((bn, h), lambda i: (i, 0)),
            pl.BlockSpec((bn, h), lambda i: (i, 0)),
            pl.BlockSpec((1, h), lambda i: (0, 0)),
        ],
        out_shape=[
            jax.ShapeDtypeStruct((n, h), jnp.float32),
            jax.ShapeDtypeStruct((n, h), jnp.float32),
            jax.ShapeDtypeStruct((1, h), jnp.float32),
        ],
        scratch_shapes=[
            pltpu.VMEM((n, 2 * h), jnp.float32),
        ],
    )(a2, x, xn, w, mat, br)


def kernel(x, x_neg, adj, diff, W1, W2, b1, b2, a, is_sparse):
    n, f = x.shape
    h = W1.shape[1]

    bn = 200
    while n % bn != 0 or bn % 8 != 0:
        bn //= 2
    a2 = jnp.reshape(a, (1,)).astype(jnp.float32)
    b1r = jnp.reshape(b1, (1, h))
    b2r = jnp.reshape(b2, (1, h))

    h1, h3, c1 = _propagate(a2, x, x_neg, W1, adj, b1r, n, f, h, bn)
    h2, h4, c2 = _propagate(a2, x, x_neg, W2, diff, b2r, n, f, h, bn)

    return (c1[0], c2[0], h1, h2, h3, h4)


# speedup vs baseline: 1.2378x; 1.0096x over previous
import functools

import jax
import jax.numpy as jnp
from jax.experimental import pallas as pl
from jax.experimental.pallas import tpu as pltpu


def _body(a_ref, x_ref, xn_ref, w1_ref, w2_ref, adj_ref, diff_ref,
          b1_ref, b2_ref,
          h1_ref, h2_ref, h3_ref, h4_ref, c1_ref, c2_ref,
          y_ref, *, n):
    j = pl.program_id(0)
    i = pl.program_id(1)
    steps = pl.num_programs(1)
    alpha = a_ref[0]
    h = h1_ref.shape[1]

    @pl.when(i == 0)
    def _():
        w = jnp.where(j == 0, w1_ref[...], w2_ref[...])
        y_ref[:, :h] = jnp.dot(x_ref[...], w,
                               preferred_element_type=jnp.float32)
        y_ref[:, h:] = jnp.dot(xn_ref[...], w,
                               preferred_element_type=jnp.float32)

    @pl.when(j == 0)
    def _():
        p = jnp.dot(adj_ref[...], y_ref[...],
                    preferred_element_type=jnp.float32)
        zp = p[:, :h] + b1_ref[...]
        zn = p[:, h:] + b1_ref[...]
        hp = jnp.where(zp >= 0, zp, alpha * zp)
        h3_ref[...] = jnp.where(zn >= 0, zn, alpha * zn)
        h1_ref[...] = hp
        s = jnp.sum(hp, axis=0, keepdims=True)

        @pl.when(i == 0)
        def _():
            c1_ref[...] = s

        @pl.when(i > 0)
        def _():
            c1_ref[...] += s

        @pl.when(i == steps - 1)
        def _():
            c1_ref[...] = jax.nn.sigmoid(c1_ref[...] * (1.0 / n))

    @pl.when(j == 1)
    def _():
        p = jnp.dot(diff_ref[...], y_ref[...],
                    preferred_element_type=jnp.float32)
        zp = p[:, :h] + b2_ref[...]
        zn = p[:, h:] + b2_ref[...]
        hp = jnp.where(zp >= 0, zp, alpha * zp)
        h4_ref[...] = jnp.where(zn >= 0, zn, alpha * zn)
        h2_ref[...] = hp
        s = jnp.sum(hp, axis=0, keepdims=True)

        @pl.when(i == 0)
        def _():
            c2_ref[...] = s

        @pl.when(i > 0)
        def _():
            c2_ref[...] += s

        @pl.when(i == steps - 1)
        def _():
            c2_ref[...] = jax.nn.sigmoid(c2_ref[...] * (1.0 / n))


def kernel(x, x_neg, adj, diff, W1, W2, b1, b2, a, is_sparse):
    n, f = x.shape
    h = W1.shape[1]

    bn = 200
    while n % bn != 0 or bn % 8 != 0:
        bn //= 2
    steps = n // bn
    last = steps - 1

    a2 = jnp.reshape(a, (1,)).astype(jnp.float32)
    b1r = jnp.reshape(b1, (1, h))
    b2r = jnp.reshape(b2, (1, h))

    def adj_side(j, i):
        return (jnp.where(j == 0, i, last), 0)

    def diff_side(j, i):
        return (jnp.where(j == 0, 0, i), 0)

    h1, h2, h3, h4, c1, c2 = pl.pallas_call(
        functools.partial(_body, n=float(n)),
        grid=(2, steps),
        in_specs=[
            pl.BlockSpec(memory_space=pltpu.SMEM),
            pl.BlockSpec((n, f), lambda j, i: (0, 0)),
            pl.BlockSpec((n, f), lambda j, i: (0, 0)),
            pl.BlockSpec((f, h), lambda j, i: (0, 0)),
            pl.BlockSpec((f, h), lambda j, i: (0, 0)),
            pl.BlockSpec((bn, n), adj_side),
            pl.BlockSpec((bn, n), diff_side),
            pl.BlockSpec((1, h), lambda j, i: (0, 0)),
            pl.BlockSpec((1, h), lambda j, i: (0, 0)),
        ],
        out_specs=[
            pl.BlockSpec((bn, h), adj_side),
            pl.BlockSpec((bn, h), diff_side),
            pl.BlockSpec((bn, h), adj_side),
            pl.BlockSpec((bn, h), diff_side),
            pl.BlockSpec((1, h), lambda j, i: (0, 0)),
            pl.BlockSpec((1, h), lambda j, i: (0, 0)),
        ],
        out_shape=[
            jax.ShapeDtypeStruct((n, h), jnp.float32),
            jax.ShapeDtypeStruct((n, h), jnp.float32),
            jax.ShapeDtypeStruct((n, h), jnp.float32),
            jax.ShapeDtypeStruct((n, h), jnp.float32),
            jax.ShapeDtypeStruct((1, h), jnp.float32),
            jax.ShapeDtypeStruct((1, h), jnp.float32),
        ],
        scratch_shapes=[
            pltpu.VMEM((n, 2 * h), jnp.float32),
        ],
    )(a2, x, x_neg, W1, W2, adj, diff, b1r, b2r)

    return (c1[0], c2[0], h1, h2, h3, h4)
